# baseline (speedup 1.0000x reference)
import functools

import jax
import jax.numpy as jnp
from jax import lax
from jax.experimental import pallas as pl
from jax.experimental.pallas import tpu as pltpu
from jax.experimental.pallas import tpu_sc as plsc

NU = 25000
NIS = 25000
NIT = 25000
D = 128
B = 4096

NC = 2
NS = 16
SUB = 128
K = 2
MBI = 4
GBI = 2
DM = 2
DG = 1

_f32 = jnp.float32


def _mesh():
    return plsc.VectorSubcoreMesh(core_axis_name="c", subcore_axis_name="s")


@functools.cache
def _spmm_builder(N, G, W, NIT):
    P = G // NC
    NR = N // NS
    ZR = 64
    nz_full, nz_rem = NR // ZR, NR % ZR
    MBYT = K * 3 * SUB * 4
    GBYT = K * SUB * W * 4

    @functools.partial(
        pl.kernel,
        out_type=(jax.ShapeDtypeStruct((G, N, W), _f32),
                  jax.ShapeDtypeStruct((8,), jnp.int32)),
        mesh=_mesh(),
        scratch_types=[
            pltpu.VMEM((MBI, K, 3, SUB), jnp.int32),
            pltpu.VMEM((GBI * K * SUB, W), _f32),
            pltpu.VMEM((ZR, W), _f32),
            pltpu.VMEM_SHARED((N, W), _f32),
            pltpu.SemaphoreType.DMA((MBI,)),
            pltpu.SemaphoreType.DMA((GBI,)),
            pltpu.SemaphoreType.DMA((GBI,)),
        ],
        compiler_params=pltpu.CompilerParams(use_tc_tiling_on_sc=False, needs_layout_passes=False),
    )
    def spmm(x_hbm, meta_hbm, tok_hbm, out_hbm, tok_out_hbm,
             meta_v, gbuf, zbuf, acc, msem, gsem, ssem):
        cid = lax.axis_index("c")
        sid = lax.axis_index("s")

        def meta_start(i):
            b = lax.rem(i, MBI)
            pltpu.async_copy(meta_hbm.at[sid, i], meta_v.at[b], msem.at[b])

        def meta_wait(i):
            b = lax.rem(i, MBI)
            pltpu.make_async_copy(meta_hbm.at[sid, i], meta_v.at[b],
                                  msem.at[b]).wait()

        def gather_start(i, g):
            bm = lax.rem(i, MBI)
            bg = lax.rem(i, GBI)
            for k in range(K):
                pltpu.async_copy(
                    x_hbm.at[g].at[meta_v.at[bm, k, 0]],
                    gbuf.at[pl.ds((bg * K + k) * SUB, SUB)], gsem.at[bg])

        def gather_wait(i, g):
            bm = lax.rem(i, MBI)
            bg = lax.rem(i, GBI)
            for k in range(K):
                pltpu.make_async_copy(
                    x_hbm.at[g].at[meta_v.at[bm, k, 0]],
                    gbuf.at[pl.ds((bg * K + k) * SUB, SUB)],
                    gsem.at[bg]).wait()

        def scatter_start(i):
            bm = lax.rem(i, MBI)
            bg = lax.rem(i, GBI)
            for k in range(K):
                pltpu.async_copy(
                    gbuf.at[pl.ds((bg * K + k) * SUB, SUB)],
                    acc.at[meta_v.at[bm, k, 1]], ssem.at[bg], add=True)

        def scatter_wait(i):
            bm = lax.rem(i, MBI)
            bg = lax.rem(i, GBI)
            for k in range(K):
                pltpu.make_async_copy(
                    gbuf.at[pl.ds((bg * K + k) * SUB, SUB)],
                    acc.at[meta_v.at[bm, k, 1]], ssem.at[bg]).wait()

        def scale(i):
            bm = lax.rem(i, MBI)
            bg = lax.rem(i, GBI)
            for k in range(K):
                gr = (bg * K + k) * SUB
                for g16 in range(SUB // 16):
                    vv = plsc.bitcast(
                        meta_v[bm, k, 2, pl.ds(g16 * 16, 16)], _f32)
                    for e in range(16):
                        v = vv[e]
                        o = g16 * 16 + e
                        for j in range(W // 16):
                            sl = pl.ds(j * 16, 16)
                            gbuf[gr + o, sl] = gbuf[gr + o, sl] * v

        @pl.loop(0, ZR)
        def _(i):
            for j in range(W // 16):
                zbuf[i, pl.ds(j * 16, 16)] = jnp.zeros((16,), _f32)

        @pl.loop(0, P)
        def _(p):
            g = cid * P + p
            base = sid * NR

            @pl.loop(0, nz_full)
            def _(i):
                pltpu.sync_copy(zbuf, acc.at[pl.ds(base + i * ZR, ZR)])

            if nz_rem:
                pltpu.sync_copy(zbuf.at[pl.ds(0, nz_rem)],
                                acc.at[pl.ds(base + nz_full * ZR, nz_rem)])
            plsc.subcore_barrier()

            for i in range(DM):
                meta_start(i)
            for i in range(DG):
                meta_wait(i)
                gather_start(i, g)

            @pl.loop(0, NIT)
            def _(i):
                @pl.when(i >= 1)
                def _():
                    scatter_wait(i - 1)

                @pl.when(i + DM < NIT)
                def _():
                    meta_start(i + DM)

                @pl.when(i + DG < NIT)
                def _():
                    meta_wait(i + DG)
                    gather_start(i + DG, g)

                gather_wait(i, g)
                scale(i)
                scatter_start(i)

            scatter_wait(NIT - 1)
            plsc.subcore_barrier()

            pltpu.sync_copy(acc.at[pl.ds(base, NR)],
                            out_hbm.at[g, pl.ds(base, NR)])
            plsc.subcore_barrier()

        @pl.when(jnp.logical_and(cid == 0, sid == 0))
        def _():
            pltpu.sync_copy(tok_hbm, tok_out_hbm)

    return spmm


_TASKS = ((0, 4, 32), (0, 4, 32), (1, 4, 32), (1, 4, 32), (2, 8, 16), (2, 8, 16))
_BT = B // (NC * NS)


@functools.cache
def _gather_mean_builder():
    out_types = [jax.ShapeDtypeStruct((g, B, w), _f32) for _, g, w in _TASKS]

    @functools.partial(
        pl.kernel,
        out_type=out_types,
        mesh=_mesh(),
        scratch_types=[
            pltpu.VMEM((_BT,), jnp.int32),
            pltpu.VMEM((_BT, 32), _f32),
            pltpu.VMEM((_BT, 32), _f32),
            pltpu.VMEM((_BT, 16), _f32),
            pltpu.VMEM((_BT, 16), _f32),
            pltpu.SemaphoreType.DMA,
        ],
        compiler_params=pltpu.CompilerParams(use_tc_tiling_on_sc=False, needs_layout_passes=False),
    )
    def gather_mean(*refs):
        embs = (refs[0:4], refs[4:8], refs[8:12])
        idxs = refs[12:18]
        outs = refs[18:24]
        idx_v, gb32, ac32, gb16, ac16, sem = refs[24:30]

        cid = lax.axis_index("c")
        sid = lax.axis_index("s")
        wid = sid * NC + cid
        base = wid * _BT

        for t, (gt, G, W) in enumerate(_TASKS):
            gb, ac = (gb32, ac32) if W == 32 else (gb16, ac16)
            pltpu.sync_copy(idxs[t].at[pl.ds(base, _BT)], idx_v)
            for g in range(G):
                for l in range(4):
                    pltpu.async_copy(
                        embs[gt][l].at[g].at[idx_v],
                        ac if l == 0 else gb, sem
                    ).wait()
                    if l > 0:
                        @pl.loop(0, _BT)
                        def _(i):
                            for j in range(W // 16):
                                sl = pl.ds(j * 16, 16)
                                ac[i, sl] = ac[i, sl] + gb[i, sl]

                @pl.loop(0, _BT)
                def _(i):
                    for j in range(W // 16):
                        sl = pl.ds(j * 16, 16)
                        ac[i, sl] = ac[i, sl] * 0.25

                pltpu.sync_copy(ac, outs[t].at[g, pl.ds(base, _BT)])

    return gather_mean


def _loss_body(a_ref, b_ref, c_ref, d_ref, e_ref, f_ref, o_ref):
    def cos(x1, x2):
        n1 = jnp.sqrt(jnp.sum(x1 * x1, axis=-1))
        n2 = jnp.sqrt(jnp.sum(x2 * x2, axis=-1))
        dot = jnp.sum(x1 * x2, axis=-1)
        return dot / jnp.maximum(n1 * n2, 1e-8)

    sp_spe, sn_spe = a_ref[...], b_ref[...]
    tp_spe, tn_spe = c_ref[...], d_ref[...]
    sp_sha, tp_sha = e_ref[...], f_ref[...]
    loss = (jnp.mean(1.0 - cos(sp_spe, sp_sha))
            + jnp.mean(jnp.maximum(cos(sn_spe, sp_sha), 0.0))
            + jnp.mean(1.0 - cos(tp_spe, tp_sha))
            + jnp.mean(jnp.maximum(cos(tn_spe, tp_sha), 0.0)))
    o_ref[0, 0] = loss


def _loss_tc(sp_spe, sn_spe, tp_spe, tn_spe, sp_sha, tp_sha):
    return pl.pallas_call(
        _loss_body,
        out_shape=jax.ShapeDtypeStruct((1, 1), _f32),
        out_specs=pl.BlockSpec(memory_space=pltpu.SMEM),
    )(sp_spe, sn_spe, tp_spe, tn_spe, sp_sha, tp_sha)


def _to_layout(x, G, W, npad):
    n = x.shape[0]
    if npad != n:
        x = jnp.concatenate([x, jnp.zeros((npad - n, D), x.dtype)])
    return x.reshape(npad, G, W).transpose(1, 0, 2)


def _prep_edges(rows, cols, vals, nit):
    epad = NS * nit * K * SUB
    pad = epad - rows.shape[0]
    rows = jnp.pad(rows, (0, pad)).reshape(NS, nit, K, 1, SUB)
    cols = jnp.pad(cols, (0, pad)).reshape(NS, nit, K, 1, SUB)
    vals = lax.bitcast_convert_type(jnp.pad(vals, (0, pad)), jnp.int32)
    vals = vals.reshape(NS, nit, K, 1, SUB)
    return jnp.concatenate([cols, rows, vals], axis=3)


def kernel(src_user_emb, tgt_user_emb, src_item_emb, tgt_item_emb,
           share_user_emb, s_rows, s_cols, s_vals, t_rows, t_cols, t_vals,
           c_rows, c_cols, c_vals, user, source_pos_item, source_neg_item,
           target_pos_item, target_neg_item, source_pop_item,
           target_pop_item):
    NSN = 50048
    NCP = 75008
    CS = 123
    CC = 147

    xs = _to_layout(jnp.concatenate([src_user_emb, src_item_emb]), 4, 32, NSN)
    xt = _to_layout(jnp.concatenate([tgt_user_emb, tgt_item_emb]), 4, 32, NSN)
    xc = _to_layout(
        jnp.concatenate([share_user_emb, src_item_emb, tgt_item_emb]),
        8, 16, NCP)

    meta_s = _prep_edges(s_rows, s_cols, s_vals, CS)
    meta_t = _prep_edges(t_rows, t_cols, t_vals, CS)
    meta_c = _prep_edges(c_rows, c_cols, c_vals, CC)

    spmm_st = _spmm_builder(NSN, 4, 32, CS)
    spmm_c = _spmm_builder(NCP, 8, 16, CC)

    es, et, ec = [xs], [xt], [xc]
    tok = jnp.zeros((8,), jnp.int32)
    for _ in range(3):
        e, tok = spmm_st(es[-1], meta_s, tok)
        es.append(e)
        e, tok = spmm_st(et[-1], meta_t, tok)
        et.append(e)
        e, tok = spmm_c(ec[-1], meta_c, tok)
        ec.append(e)

    idx_sp_s = NU + source_pos_item
    idx_sn_s = NU + source_neg_item
    idx_tp_t = NU + target_pos_item
    idx_tn_t = NU + target_neg_item
    idx_sp_c = NU + source_pos_item
    idx_tp_c = NU + NIS + target_pos_item

    outs = _gather_mean_builder()(
        *es, *et, *ec,
        idx_sp_s, idx_sn_s, idx_tp_t, idx_tn_t, idx_sp_c, idx_tp_c)
    flat = [o.transpose(1, 0, 2).reshape(B, D) for o in outs]
    loss = _loss_tc(*flat)
    return loss[0, 0]


# speedup vs baseline: 1.1874x; 1.1874x over previous
import functools

import jax
import jax.numpy as jnp
from jax import lax
from jax.experimental import pallas as pl
from jax.experimental.pallas import tpu as pltpu
from jax.experimental.pallas import tpu_sc as plsc

NU = 25000
NIS = 25000
NIT = 25000
D = 128
B = 4096

NC = 2
NS = 16
SUB = 128
K = 1
MBI = 8
GBI = 6
DM = 6
DG = 4

_f32 = jnp.float32


def _mesh():
    return plsc.VectorSubcoreMesh(core_axis_name="c", subcore_axis_name="s")


@functools.cache
def _spmm_builder(N, G, W, NIT):
    P = G // NC
    NR = N // NS
    ZR = 64
    nz_full, nz_rem = NR // ZR, NR % ZR
    MBYT = K * 3 * SUB * 4
    GBYT = K * SUB * W * 4

    @functools.partial(
        pl.kernel,
        out_type=(jax.ShapeDtypeStruct((G, N, W), _f32),
                  jax.ShapeDtypeStruct((8,), jnp.int32)),
        mesh=_mesh(),
        scratch_types=[
            pltpu.VMEM((MBI, K, 3, SUB), jnp.int32),
            pltpu.VMEM((GBI * K * SUB, W), _f32),
            pltpu.VMEM((ZR, W), _f32),
            pltpu.VMEM_SHARED((N, W), _f32),
            pltpu.SemaphoreType.DMA((MBI,)),
            pltpu.SemaphoreType.DMA((GBI,)),
            pltpu.SemaphoreType.DMA((GBI,)),
        ],
        compiler_params=pltpu.CompilerParams(use_tc_tiling_on_sc=False, needs_layout_passes=False),
    )
    def spmm(x_hbm, meta_hbm, tok_hbm, out_hbm, tok_out_hbm,
             meta_v, gbuf, zbuf, acc, msem, gsem, ssem):
        cid = lax.axis_index("c")
        sid = lax.axis_index("s")

        def meta_start(i):
            b = lax.rem(i, MBI)
            pltpu.async_copy(meta_hbm.at[sid, i], meta_v.at[b], msem.at[b])

        def meta_wait(i):
            b = lax.rem(i, MBI)
            pltpu.make_async_copy(meta_hbm.at[sid, i], meta_v.at[b],
                                  msem.at[b]).wait()

        def gather_start(i, g):
            bm = lax.rem(i, MBI)
            bg = lax.rem(i, GBI)
            for k in range(K):
                pltpu.async_copy(
                    x_hbm.at[g].at[meta_v.at[bm, k, 0]],
                    gbuf.at[pl.ds((bg * K + k) * SUB, SUB)], gsem.at[bg])

        def gather_wait(i, g):
            bg = lax.rem(i, GBI)
            for k in range(K):
                pltpu.make_async_copy(
                    out_hbm.at[g, pl.ds(0, SUB)],
                    gbuf.at[pl.ds((bg * K + k) * SUB, SUB)],
                    gsem.at[bg]).wait()

        def scatter_start(i):
            bm = lax.rem(i, MBI)
            bg = lax.rem(i, GBI)
            for k in range(K):
                pltpu.async_copy(
                    gbuf.at[pl.ds((bg * K + k) * SUB, SUB)],
                    acc.at[meta_v.at[bm, k, 1]], ssem.at[bg], add=True)

        def scatter_wait(i, g):
            bg = lax.rem(i, GBI)
            for k in range(K):
                pltpu.make_async_copy(
                    gbuf.at[pl.ds((bg * K + k) * SUB, SUB)],
                    out_hbm.at[g, pl.ds(0, SUB)], ssem.at[bg]).wait()

        def scale(i):
            bm = lax.rem(i, MBI)
            bg = lax.rem(i, GBI)
            for k in range(K):
                gr = (bg * K + k) * SUB
                for g16 in range(SUB // 16):
                    vv = plsc.bitcast(
                        meta_v[bm, k, 2, pl.ds(g16 * 16, 16)], _f32)
                    for e in range(16):
                        v = vv[e]
                        o = g16 * 16 + e
                        for j in range(W // 16):
                            sl = pl.ds(j * 16, 16)
                            gbuf[gr + o, sl] = gbuf[gr + o, sl] * v

        @pl.loop(0, ZR)
        def _(i):
            for j in range(W // 16):
                zbuf[i, pl.ds(j * 16, 16)] = jnp.zeros((16,), _f32)

        @pl.loop(0, P)
        def _(p):
            g = cid * P + p
            base = sid * NR

            @pl.loop(0, nz_full)
            def _(i):
                pltpu.sync_copy(zbuf, acc.at[pl.ds(base + i * ZR, ZR)])

            if nz_rem:
                pltpu.sync_copy(zbuf.at[pl.ds(0, nz_rem)],
                                acc.at[pl.ds(base + nz_full * ZR, nz_rem)])
            plsc.subcore_barrier()

            for i in range(DM):
                meta_start(i)
            for i in range(DG):
                meta_wait(i)
                gather_start(i, g)

            @pl.loop(0, NIT)
            def _(i):
                @pl.when(i >= 2)
                def _():
                    scatter_wait(i - 2, g)

                @pl.when(i + DM < NIT)
                def _():
                    meta_start(i + DM)

                @pl.when(i + DG < NIT)
                def _():
                    meta_wait(i + DG)
                    gather_start(i + DG, g)

                gather_wait(i, g)
                scale(i)
                scatter_start(i)

            scatter_wait(NIT - 2, g)
            scatter_wait(NIT - 1, g)
            plsc.subcore_barrier()

            pltpu.sync_copy(acc.at[pl.ds(base, NR)],
                            out_hbm.at[g, pl.ds(base, NR)])
            plsc.subcore_barrier()

        @pl.when(jnp.logical_and(cid == 0, sid == 0))
        def _():
            pltpu.sync_copy(tok_hbm, tok_out_hbm)

    return spmm


_TASKS = ((0, 4, 32), (0, 4, 32), (1, 4, 32), (1, 4, 32), (2, 8, 16), (2, 8, 16))
_BT = B // (NC * NS)


@functools.cache
def _gather_mean_builder():
    out_types = [jax.ShapeDtypeStruct((g, B, w), _f32) for _, g, w in _TASKS]

    @functools.partial(
        pl.kernel,
        out_type=out_types,
        mesh=_mesh(),
        scratch_types=[
            pltpu.VMEM((_BT,), jnp.int32),
            pltpu.VMEM((_BT, 32), _f32),
            pltpu.VMEM((_BT, 32), _f32),
            pltpu.VMEM((_BT, 16), _f32),
            pltpu.VMEM((_BT, 16), _f32),
            pltpu.SemaphoreType.DMA,
        ],
        compiler_params=pltpu.CompilerParams(use_tc_tiling_on_sc=False, needs_layout_passes=False),
    )
    def gather_mean(*refs):
        embs = (refs[0:4], refs[4:8], refs[8:12])
        idxs = refs[12:18]
        outs = refs[18:24]
        idx_v, gb32, ac32, gb16, ac16, sem = refs[24:30]

        cid = lax.axis_index("c")
        sid = lax.axis_index("s")
        wid = sid * NC + cid
        base = wid * _BT

        for t, (gt, G, W) in enumerate(_TASKS):
            gb, ac = (gb32, ac32) if W == 32 else (gb16, ac16)
            pltpu.sync_copy(idxs[t].at[pl.ds(base, _BT)], idx_v)
            for g in range(G):
                for l in range(4):
                    pltpu.async_copy(
                        embs[gt][l].at[g].at[idx_v],
                        ac if l == 0 else gb, sem
                    ).wait()
                    if l > 0:
                        @pl.loop(0, _BT)
                        def _(i):
                            for j in range(W // 16):
                                sl = pl.ds(j * 16, 16)
                                ac[i, sl] = ac[i, sl] + gb[i, sl]

                @pl.loop(0, _BT)
                def _(i):
                    for j in range(W // 16):
                        sl = pl.ds(j * 16, 16)
                        ac[i, sl] = ac[i, sl] * 0.25

                pltpu.sync_copy(ac, outs[t].at[g, pl.ds(base, _BT)])

    return gather_mean


def _loss_body(a_ref, b_ref, c_ref, d_ref, e_ref, f_ref, o_ref):
    def cos(x1, x2):
        n1 = jnp.sqrt(jnp.sum(x1 * x1, axis=-1))
        n2 = jnp.sqrt(jnp.sum(x2 * x2, axis=-1))
        dot = jnp.sum(x1 * x2, axis=-1)
        return dot / jnp.maximum(n1 * n2, 1e-8)

    sp_spe, sn_spe = a_ref[...], b_ref[...]
    tp_spe, tn_spe = c_ref[...], d_ref[...]
    sp_sha, tp_sha = e_ref[...], f_ref[...]
    loss = (jnp.mean(1.0 - cos(sp_spe, sp_sha))
            + jnp.mean(jnp.maximum(cos(sn_spe, sp_sha), 0.0))
            + jnp.mean(1.0 - cos(tp_spe, tp_sha))
            + jnp.mean(jnp.maximum(cos(tn_spe, tp_sha), 0.0)))
    o_ref[0, 0] = loss


def _loss_tc(sp_spe, sn_spe, tp_spe, tn_spe, sp_sha, tp_sha):
    return pl.pallas_call(
        _loss_body,
        out_shape=jax.ShapeDtypeStruct((1, 1), _f32),
        out_specs=pl.BlockSpec(memory_space=pltpu.SMEM),
    )(sp_spe, sn_spe, tp_spe, tn_spe, sp_sha, tp_sha)


def _to_layout(x, G, W, npad):
    n = x.shape[0]
    if npad != n:
        x = jnp.concatenate([x, jnp.zeros((npad - n, D), x.dtype)])
    return x.reshape(npad, G, W).transpose(1, 0, 2)


def _prep_edges(rows, cols, vals, nit):
    epad = NS * nit * K * SUB
    pad = epad - rows.shape[0]
    rows = jnp.pad(rows, (0, pad)).reshape(NS, nit, K, 1, SUB)
    cols = jnp.pad(cols, (0, pad)).reshape(NS, nit, K, 1, SUB)
    vals = lax.bitcast_convert_type(jnp.pad(vals, (0, pad)), jnp.int32)
    vals = vals.reshape(NS, nit, K, 1, SUB)
    return jnp.concatenate([cols, rows, vals], axis=3)


def kernel(src_user_emb, tgt_user_emb, src_item_emb, tgt_item_emb,
           share_user_emb, s_rows, s_cols, s_vals, t_rows, t_cols, t_vals,
           c_rows, c_cols, c_vals, user, source_pos_item, source_neg_item,
           target_pos_item, target_neg_item, source_pop_item,
           target_pop_item):
    NSN = 50048
    NCP = 75008
    CS = 246
    CC = 294

    xs = _to_layout(jnp.concatenate([src_user_emb, src_item_emb]), 4, 32, NSN)
    xt = _to_layout(jnp.concatenate([tgt_user_emb, tgt_item_emb]), 4, 32, NSN)
    xc = _to_layout(
        jnp.concatenate([share_user_emb, src_item_emb, tgt_item_emb]),
        8, 16, NCP)

    meta_s = _prep_edges(s_rows, s_cols, s_vals, CS)
    meta_t = _prep_edges(t_rows, t_cols, t_vals, CS)
    meta_c = _prep_edges(c_rows, c_cols, c_vals, CC)

    spmm_st = _spmm_builder(NSN, 4, 32, CS)
    spmm_c = _spmm_builder(NCP, 8, 16, CC)

    es, et, ec = [xs], [xt], [xc]
    tok = jnp.zeros((8,), jnp.int32)
    for _ in range(3):
        e, tok = spmm_st(es[-1], meta_s, tok)
        es.append(e)
        e, tok = spmm_st(et[-1], meta_t, tok)
        et.append(e)
        e, tok = spmm_c(ec[-1], meta_c, tok)
        ec.append(e)

    idx_sp_s = NU + source_pos_item
    idx_sn_s = NU + source_neg_item
    idx_tp_t = NU + target_pos_item
    idx_tn_t = NU + target_neg_item
    idx_sp_c = NU + source_pos_item
    idx_tp_c = NU + NIS + target_pos_item

    outs = _gather_mean_builder()(
        *es, *et, *ec,
        idx_sp_s, idx_sn_s, idx_tp_t, idx_tn_t, idx_sp_c, idx_tp_c)
    flat = [o.transpose(1, 0, 2).reshape(B, D) for o in outs]
    loss = _loss_tc(*flat)
    return loss[0, 0]


# speedup vs baseline: 1.1874x; 1.0000x over previous
import functools

import jax
import jax.numpy as jnp
from jax import lax
from jax.experimental import pallas as pl
from jax.experimental.pallas import tpu as pltpu
from jax.experimental.pallas import tpu_sc as plsc

NU = 25000
NIS = 25000
NIT = 25000
D = 128
B = 4096

NC = 2
NS = 16
SUB = 128
K = 1
MBI = 8
GBI = 6
DM = 6
DG = 4

_f32 = jnp.float32


def _mesh():
    return plsc.VectorSubcoreMesh(core_axis_name="c", subcore_axis_name="s")


@functools.cache
def _spmm_builder(N, G, W, NIT):
    P = G // NC
    NR = N // NS
    ZR = 64
    nz_full, nz_rem = NR // ZR, NR % ZR
    MBYT = K * 3 * SUB * 4
    GBYT = K * SUB * W * 4

    @functools.partial(
        pl.kernel,
        out_type=(jax.ShapeDtypeStruct((G, N, W), _f32),
                  jax.ShapeDtypeStruct((8,), jnp.int32)),
        mesh=_mesh(),
        scratch_types=[
            pltpu.VMEM((MBI, K, 3, SUB), jnp.int32),
            pltpu.VMEM((GBI * K * SUB, W), _f32),
            pltpu.VMEM((ZR, W), _f32),
            pltpu.VMEM_SHARED((N, W), _f32),
            pltpu.SemaphoreType.DMA((MBI,)),
            pltpu.SemaphoreType.DMA((GBI,)),
            pltpu.SemaphoreType.DMA((GBI,)),
        ],
        compiler_params=pltpu.CompilerParams(use_tc_tiling_on_sc=False, needs_layout_passes=False),
    )
    def spmm(x_hbm, meta_hbm, tok_hbm, out_hbm, tok_out_hbm,
             meta_v, gbuf, zbuf, acc, msem, gsem, ssem):
        cid = lax.axis_index("c")
        sid = lax.axis_index("s")

        def meta_start(i):
            b = lax.rem(i, MBI)
            pltpu.async_copy(meta_hbm.at[sid, i], meta_v.at[b], msem.at[b])

        def meta_wait(i):
            b = lax.rem(i, MBI)
            pltpu.make_async_copy(meta_hbm.at[sid, i], meta_v.at[b],
                                  msem.at[b]).wait()

        def gather_start(i, g):
            bm = lax.rem(i, MBI)
            bg = lax.rem(i, GBI)
            for k in range(K):
                pltpu.async_copy(
                    x_hbm.at[g].at[meta_v.at[bm, k, 0]],
                    gbuf.at[pl.ds((bg * K + k) * SUB, SUB)], gsem.at[bg])

        def gather_wait(i, g):
            bm = lax.rem(i, MBI)
            bg = lax.rem(i, GBI)
            for k in range(K):
                pltpu.make_async_copy(
                    x_hbm.at[g].at[meta_v.at[bm, k, 0]],
                    gbuf.at[pl.ds((bg * K + k) * SUB, SUB)],
                    gsem.at[bg]).wait()

        def scatter_start(i):
            bm = lax.rem(i, MBI)
            bg = lax.rem(i, GBI)
            for k in range(K):
                pltpu.async_copy(
                    gbuf.at[pl.ds((bg * K + k) * SUB, SUB)],
                    acc.at[meta_v.at[bm, k, 1]], ssem.at[bg], add=True)

        def scatter_wait(i, g):
            bm = lax.rem(i, MBI)
            bg = lax.rem(i, GBI)
            for k in range(K):
                pltpu.make_async_copy(
                    gbuf.at[pl.ds((bg * K + k) * SUB, SUB)],
                    acc.at[meta_v.at[bm, k, 1]], ssem.at[bg]).wait()

        def scale(i):
            bm = lax.rem(i, MBI)
            bg = lax.rem(i, GBI)
            for k in range(K):
                gr = (bg * K + k) * SUB
                for g16 in range(SUB // 16):
                    vv = plsc.bitcast(
                        meta_v[bm, k, 2, pl.ds(g16 * 16, 16)], _f32)
                    for e in range(16):
                        v = vv[e]
                        o = g16 * 16 + e
                        for j in range(W // 16):
                            sl = pl.ds(j * 16, 16)
                            gbuf[gr + o, sl] = gbuf[gr + o, sl] * v

        @pl.loop(0, ZR)
        def _(i):
            for j in range(W // 16):
                zbuf[i, pl.ds(j * 16, 16)] = jnp.zeros((16,), _f32)

        @pl.loop(0, P)
        def _(p):
            g = cid * P + p
            base = sid * NR

            @pl.loop(0, nz_full)
            def _(i):
                pltpu.sync_copy(zbuf, acc.at[pl.ds(base + i * ZR, ZR)])

            if nz_rem:
                pltpu.sync_copy(zbuf.at[pl.ds(0, nz_rem)],
                                acc.at[pl.ds(base + nz_full * ZR, nz_rem)])
            plsc.subcore_barrier()

            for i in range(DM):
                meta_start(i)
            for i in range(DG):
                meta_wait(i)
                gather_start(i, g)

            @pl.loop(0, NIT)
            def _(i):
                @pl.when(i >= 2)
                def _():
                    scatter_wait(i - 2, g)

                @pl.when(i + DM < NIT)
                def _():
                    meta_start(i + DM)

                @pl.when(i + DG < NIT)
                def _():
                    meta_wait(i + DG)
                    gather_start(i + DG, g)

                gather_wait(i, g)
                scale(i)
                scatter_start(i)

            scatter_wait(NIT - 2, g)
            scatter_wait(NIT - 1, g)
            plsc.subcore_barrier()

            pltpu.sync_copy(acc.at[pl.ds(base, NR)],
                            out_hbm.at[g, pl.ds(base, NR)])
            plsc.subcore_barrier()

        @pl.when(jnp.logical_and(cid == 0, sid == 0))
        def _():
            pltpu.sync_copy(tok_hbm, tok_out_hbm)

    return spmm


_TASKS = ((0, 4, 32), (0, 4, 32), (1, 4, 32), (1, 4, 32), (2, 8, 16), (2, 8, 16))
_BT = B // (NC * NS)


@functools.cache
def _gather_mean_builder():
    out_types = [jax.ShapeDtypeStruct((g, B, w), _f32) for _, g, w in _TASKS]

    @functools.partial(
        pl.kernel,
        out_type=out_types,
        mesh=_mesh(),
        scratch_types=[
            pltpu.VMEM((_BT,), jnp.int32),
            pltpu.VMEM((_BT, 32), _f32),
            pltpu.VMEM((_BT, 32), _f32),
            pltpu.VMEM((_BT, 16), _f32),
            pltpu.VMEM((_BT, 16), _f32),
            pltpu.SemaphoreType.DMA,
        ],
        compiler_params=pltpu.CompilerParams(use_tc_tiling_on_sc=False, needs_layout_passes=False),
    )
    def gather_mean(*refs):
        embs = (refs[0:4], refs[4:8], refs[8:12])
        idxs = refs[12:18]
        outs = refs[18:24]
        idx_v, gb32, ac32, gb16, ac16, sem = refs[24:30]

        cid = lax.axis_index("c")
        sid = lax.axis_index("s")
        wid = sid * NC + cid
        base = wid * _BT

        for t, (gt, G, W) in enumerate(_TASKS):
            gb, ac = (gb32, ac32) if W == 32 else (gb16, ac16)
            pltpu.sync_copy(idxs[t].at[pl.ds(base, _BT)], idx_v)
            for g in range(G):
                for l in range(4):
                    pltpu.async_copy(
                        embs[gt][l].at[g].at[idx_v],
                        ac if l == 0 else gb, sem
                    ).wait()
                    if l > 0:
                        @pl.loop(0, _BT)
                        def _(i):
                            for j in range(W // 16):
                                sl = pl.ds(j * 16, 16)
                                ac[i, sl] = ac[i, sl] + gb[i, sl]

                @pl.loop(0, _BT)
                def _(i):
                    for j in range(W // 16):
                        sl = pl.ds(j * 16, 16)
                        ac[i, sl] = ac[i, sl] * 0.25

                pltpu.sync_copy(ac, outs[t].at[g, pl.ds(base, _BT)])

    return gather_mean


def _loss_body(a_ref, b_ref, c_ref, d_ref, e_ref, f_ref, o_ref):
    def cos(x1, x2):
        n1 = jnp.sqrt(jnp.sum(x1 * x1, axis=-1))
        n2 = jnp.sqrt(jnp.sum(x2 * x2, axis=-1))
        dot = jnp.sum(x1 * x2, axis=-1)
        return dot / jnp.maximum(n1 * n2, 1e-8)

    sp_spe, sn_spe = a_ref[...], b_ref[...]
    tp_spe, tn_spe = c_ref[...], d_ref[...]
    sp_sha, tp_sha = e_ref[...], f_ref[...]
    loss = (jnp.mean(1.0 - cos(sp_spe, sp_sha))
            + jnp.mean(jnp.maximum(cos(sn_spe, sp_sha), 0.0))
            + jnp.mean(1.0 - cos(tp_spe, tp_sha))
            + jnp.mean(jnp.maximum(cos(tn_spe, tp_sha), 0.0)))
    o_ref[0, 0] = loss


def _loss_tc(sp_spe, sn_spe, tp_spe, tn_spe, sp_sha, tp_sha):
    return pl.pallas_call(
        _loss_body,
        out_shape=jax.ShapeDtypeStruct((1, 1), _f32),
        out_specs=pl.BlockSpec(memory_space=pltpu.SMEM),
    )(sp_spe, sn_spe, tp_spe, tn_spe, sp_sha, tp_sha)


def _to_layout(x, G, W, npad):
    n = x.shape[0]
    if npad != n:
        x = jnp.concatenate([x, jnp.zeros((npad - n, D), x.dtype)])
    return x.reshape(npad, G, W).transpose(1, 0, 2)


def _prep_edges(rows, cols, vals, nit):
    epad = NS * nit * K * SUB
    pad = epad - rows.shape[0]
    rows = jnp.pad(rows, (0, pad)).reshape(NS, nit, K, 1, SUB)
    cols = jnp.pad(cols, (0, pad)).reshape(NS, nit, K, 1, SUB)
    vals = lax.bitcast_convert_type(jnp.pad(vals, (0, pad)), jnp.int32)
    vals = vals.reshape(NS, nit, K, 1, SUB)
    return jnp.concatenate([cols, rows, vals], axis=3)


def kernel(src_user_emb, tgt_user_emb, src_item_emb, tgt_item_emb,
           share_user_emb, s_rows, s_cols, s_vals, t_rows, t_cols, t_vals,
           c_rows, c_cols, c_vals, user, source_pos_item, source_neg_item,
           target_pos_item, target_neg_item, source_pop_item,
           target_pop_item):
    NSN = 50048
    NCP = 75008
    CS = 246
    CC = 294

    xs = _to_layout(jnp.concatenate([src_user_emb, src_item_emb]), 4, 32, NSN)
    xt = _to_layout(jnp.concatenate([tgt_user_emb, tgt_item_emb]), 4, 32, NSN)
    xc = _to_layout(
        jnp.concatenate([share_user_emb, src_item_emb, tgt_item_emb]),
        8, 16, NCP)

    meta_s = _prep_edges(s_rows, s_cols, s_vals, CS)
    meta_t = _prep_edges(t_rows, t_cols, t_vals, CS)
    meta_c = _prep_edges(c_rows, c_cols, c_vals, CC)

    spmm_st = _spmm_builder(NSN, 4, 32, CS)
    spmm_c = _spmm_builder(NCP, 8, 16, CC)

    es, et, ec = [xs], [xt], [xc]
    tok = jnp.zeros((8,), jnp.int32)
    for _ in range(3):
        e, tok = spmm_st(es[-1], meta_s, tok)
        es.append(e)
        e, tok = spmm_st(et[-1], meta_t, tok)
        et.append(e)
        e, tok = spmm_c(ec[-1], meta_c, tok)
        ec.append(e)

    idx_sp_s = NU + source_pos_item
    idx_sn_s = NU + source_neg_item
    idx_tp_t = NU + target_pos_item
    idx_tn_t = NU + target_neg_item
    idx_sp_c = NU + source_pos_item
    idx_tp_c = NU + NIS + target_pos_item

    outs = _gather_mean_builder()(
        *es, *et, *ec,
        idx_sp_s, idx_sn_s, idx_tp_t, idx_tn_t, idx_sp_c, idx_tp_c)
    flat = [o.transpose(1, 0, 2).reshape(B, D) for o in outs]
    loss = _loss_tc(*flat)
    return loss[0, 0]
